# baseline (speedup 1.0000x reference)
import functools
import math

import jax
import jax.numpy as jnp
import numpy as np
from jax import lax
from jax.experimental import pallas as pl
from jax.experimental.pallas import tpu as pltpu
from jax.experimental.pallas import tpu_sc as plsc

VOCAB = 1000000
EMB = 64
SEQ = 200
BATCH = 4096

NUM_CORES = 2
NUM_SUBCORES = 16
NUM_WORKERS = NUM_CORES * NUM_SUBCORES

CHUNK = 128
TOTAL = BATCH * SEQ
PER_W = TOTAL // NUM_WORKERS
NCHUNK = PER_W // CHUNK
NBUF = 4
SCALE = math.sqrt(float(EMB))
PE_ROWS = 2 * SEQ

VBLK = 128
FULL_BLOCKS = VOCAB // VBLK
TAIL = VOCAB - FULL_BLOCKS * VBLK
BLK_PER_W = FULL_BLOCKS // NUM_WORKERS
BLK_EXTRA = FULL_BLOCKS - BLK_PER_W * NUM_WORKERS
SKEW = VBLK + 1


def _positional_encoding_np(max_len, d):
    position = np.arange(max_len)[:, None].astype(np.float32)
    div_term = np.exp(np.arange(0, d, 2).astype(np.float32) * (-np.log(10000.0) / d))
    pe = np.zeros((max_len, d), dtype=np.float32)
    pe[:, 0::2] = np.sin(position * div_term)
    pe[:, 1::2] = np.cos(position * div_term)
    return pe


_PE2 = np.concatenate([_positional_encoding_np(SEQ, EMB)] * 2, axis=0).reshape(-1)


def _transpose_kernel(tt_hbm, tail_hbm, out_hbm, in_s, buf, tail_v, sem_i, sem_o):
    wid = lax.axis_index("s") * NUM_CORES + lax.axis_index("c")
    start = BLK_PER_W * wid + jnp.minimum(wid, BLK_EXTRA)
    count = BLK_PER_W + jnp.where(wid < BLK_EXTRA, 1, 0)
    iota = lax.iota(jnp.int32, 16)

    def blk(i, _):
        r0 = (start + i) * VBLK
        pltpu.async_copy(tt_hbm.at[:, pl.ds(r0, VBLK)],
                         in_s.at[:, pl.ds(0, VBLK)], sem_i).wait()

        @plsc.parallel_loop(0, VBLK, unroll=2)
        def row(r):
            rv = jnp.full((16,), r, dtype=jnp.int32)
            for d in range(EMB // 16):
                v = plsc.load_gather(in_s, [iota + d * 16, rv])
                buf[pl.ds(r * EMB + d * 16, 16)] = v

        pltpu.async_copy(buf, out_hbm.at[pl.ds(r0 * EMB, VBLK * EMB)],
                         sem_o).wait()
        return 0

    lax.fori_loop(0, count, blk, 0)

    @pl.when(wid == NUM_WORKERS - 1)
    def _():
        pltpu.async_copy(tail_hbm, tail_v, sem_i).wait()

        @plsc.parallel_loop(0, TAIL, unroll=2)
        def row(r):
            for d in range(EMB // 16):
                buf[pl.ds(r * EMB + d * 16, 16)] = tail_v[r, pl.ds(d * 16, 16)]

        pltpu.async_copy(buf.at[pl.ds(0, TAIL * EMB)],
                         out_hbm.at[pl.ds(FULL_BLOCKS * VBLK * EMB, TAIL * EMB)],
                         sem_o).wait()


def _sc_kernel(x_hbm, table_hbm, pe_hbm, out_hbm,
               idx_v, in0, in1, in2, in3, ou0, ou1, ou2, ou3, pe_v,
               sg0, sg1, sg2, sg3, ss0, ss1, ss2, ss3):
    ins = (in0, in1, in2, in3)
    outs = (ou0, ou1, ou2, ou3)
    sgs = (sg0, sg1, sg2, sg3)
    sss = (ss0, ss1, ss2, ss3)

    wid = lax.axis_index("s") * NUM_CORES + lax.axis_index("c")
    pltpu.sync_copy(pe_hbm, pe_v)
    pltpu.sync_copy(x_hbm.at[pl.ds(wid * NCHUNK, NCHUNK)], idx_v)
    base = wid * PER_W

    def g_cp(c, b):
        return pltpu.make_async_copy(table_hbm.at[idx_v.at[c]], ins[b], sgs[b])

    def s_cp(c, b):
        dst = out_hbm.at[pl.ds((base + c * CHUNK) * EMB, CHUNK * EMB)]
        return pltpu.make_async_copy(outs[b], dst, sss[b])

    for b in range(NBUF):
        g_cp(b, b).start()

    def quad(g, _):
        for b in range(NBUF):
            c = NBUF * g + b
            g_cp(c, b).wait()

            @pl.when(g >= 1)
            def _():
                s_cp(c - NBUF, b).wait()

            p0 = lax.rem(c * CHUNK, SEQ) * EMB
            i_buf = ins[b]
            o_buf = outs[b]

            @plsc.parallel_loop(0, CHUNK, unroll=4)
            def row(r):
                for d in range(EMB // 16):
                    o_buf[pl.ds(r * EMB + d * 16, 16)] = (
                        i_buf[r, pl.ds(d * 16, 16)] * SCALE
                        + pe_v[pl.ds(p0 + r * EMB + d * 16, 16)])

            @pl.when(g < NCHUNK // NBUF - 1)
            def _():
                g_cp(c + NBUF, b).start()

            s_cp(c, b).start()
        return 0

    lax.fori_loop(0, NCHUNK // NBUF, quad, 0)
    for b in range(NBUF):
        s_cp(NCHUNK - NBUF + b, b).wait()


def _impl(x, table):
    mesh = plsc.VectorSubcoreMesh(core_axis_name="c", subcore_axis_name="s")

    t_lin = pl.kernel(
        _transpose_kernel,
        out_type=jax.ShapeDtypeStruct((VOCAB * EMB,), jnp.float32),
        mesh=mesh,
        compiler_params=pltpu.CompilerParams(use_tc_tiling_on_sc=True,
                                             needs_layout_passes=False),
        scratch_types=[
            pltpu.VMEM((EMB, SKEW), jnp.float32),
            pltpu.VMEM((VBLK * EMB,), jnp.float32),
            pltpu.VMEM((TAIL, EMB), jnp.float32),
            pltpu.SemaphoreType.DMA,
            pltpu.SemaphoreType.DMA,
        ],
    )(table.T, table[VOCAB - TAIL:])

    xf = x.reshape(TOTAL // CHUNK, CHUNK)
    out = pl.kernel(
        _sc_kernel,
        out_type=jax.ShapeDtypeStruct((TOTAL * EMB,), jnp.float32),
        mesh=mesh,
        compiler_params=pltpu.CompilerParams(use_tc_tiling_on_sc=False),
        scratch_types=(
            [pltpu.VMEM((NCHUNK, CHUNK), jnp.int32)]
            + [pltpu.VMEM((CHUNK, EMB), jnp.float32) for _ in range(NBUF)]
            + [pltpu.VMEM((CHUNK * EMB,), jnp.float32) for _ in range(NBUF)]
            + [pltpu.VMEM((PE_ROWS * EMB,), jnp.float32)]
            + [pltpu.SemaphoreType.DMA] * (2 * NBUF)
        ),
    )(xf, t_lin.reshape(VOCAB, EMB), jnp.asarray(_PE2))
    return out.reshape(BATCH, SEQ, EMB)


def kernel(x, table):
    return _impl(x, table)


# speedup vs baseline: 1.2469x; 1.2469x over previous
import functools
import math

import jax
import jax.numpy as jnp
import numpy as np
from jax import lax
from jax.experimental import pallas as pl
from jax.experimental.pallas import tpu as pltpu
from jax.experimental.pallas import tpu_sc as plsc

VOCAB = 1000000
EMB = 64
SEQ = 200
BATCH = 4096

NUM_CORES = 2
NUM_SUBCORES = 16
NUM_WORKERS = NUM_CORES * NUM_SUBCORES

CHUNK = 128
TOTAL = BATCH * SEQ
PER_W = TOTAL // NUM_WORKERS
NCHUNK = PER_W // CHUNK
NBUF = 4
SCALE = math.sqrt(float(EMB))
PE_ROWS = 2 * SEQ

VBLK = 128
FULL_BLOCKS = VOCAB // VBLK
TAIL = VOCAB - FULL_BLOCKS * VBLK
BLK_PER_W = FULL_BLOCKS // NUM_WORKERS
BLK_EXTRA = FULL_BLOCKS - BLK_PER_W * NUM_WORKERS
SKEW = VBLK + 1


def _positional_encoding_np(max_len, d):
    position = np.arange(max_len)[:, None].astype(np.float32)
    div_term = np.exp(np.arange(0, d, 2).astype(np.float32) * (-np.log(10000.0) / d))
    pe = np.zeros((max_len, d), dtype=np.float32)
    pe[:, 0::2] = np.sin(position * div_term)
    pe[:, 1::2] = np.cos(position * div_term)
    return pe


_PE2 = np.concatenate([_positional_encoding_np(SEQ, EMB)] * 2, axis=0).reshape(-1)


def _transpose_kernel(tt_hbm, tail_hbm, out_hbm,
                      s0, s1, b0, b1, tail_v, si0, si1, so0, so1):
    ss = (s0, s1)
    bs = (b0, b1)
    sis = (si0, si1)
    sos = (so0, so1)
    wid = lax.axis_index("s") * NUM_CORES + lax.axis_index("c")
    start = BLK_PER_W * wid
    iota = lax.iota(jnp.int32, 16)

    def i_cp(blk_idx, b):
        src = tt_hbm.at[:, pl.ds(blk_idx * VBLK, VBLK)]
        return pltpu.make_async_copy(src, ss[b].at[:, pl.ds(0, VBLK)], sis[b])

    def o_cp(blk_idx, b):
        dst = out_hbm.at[pl.ds(blk_idx * VBLK * EMB, VBLK * EMB)]
        return pltpu.make_async_copy(bs[b], dst, sos[b])

    def compute(b):
        in_s = ss[b]
        buf = bs[b]

        @plsc.parallel_loop(0, VBLK, unroll=4)
        def row(r):
            rv = jnp.full((16,), r, dtype=jnp.int32)
            for d in range(EMB // 16):
                v = plsc.load_gather(in_s, [iota + d * 16, rv])
                buf[pl.ds(r * EMB + d * 16, 16)] = v

    i_cp(start, 0).start()

    def pair(g, _):
        for b in range(2):
            i = 2 * g + b
            i_cp(start + i, b).wait()

            cond = (g < BLK_PER_W // 2 - 1) if b else True
            if b == 0:
                i_cp(start + i + 1, 1).start()
            else:
                @pl.when(cond)
                def _():
                    i_cp(start + i + 1, 0).start()

            @pl.when(g >= 1)
            def _():
                o_cp(start + i - 2, b).wait()

            compute(b)
            o_cp(start + i, b).start()
        return 0

    lax.fori_loop(0, BLK_PER_W // 2, pair, 0)
    o_cp(start + BLK_PER_W - 2, 0).wait()
    o_cp(start + BLK_PER_W - 1, 1).wait()

    @pl.when(wid < BLK_EXTRA)
    def _():
        extra = NUM_WORKERS * BLK_PER_W + wid
        i_cp(extra, 0).start()
        i_cp(extra, 0).wait()
        compute(0)
        o_cp(extra, 0).start()
        o_cp(extra, 0).wait()

    @pl.when(wid == NUM_WORKERS - 1)
    def _():
        pltpu.async_copy(tail_hbm, tail_v, si0).wait()

        @plsc.parallel_loop(0, TAIL, unroll=2)
        def row(r):
            for d in range(EMB // 16):
                b0[pl.ds(r * EMB + d * 16, 16)] = tail_v[r, pl.ds(d * 16, 16)]

        pltpu.async_copy(b0.at[pl.ds(0, TAIL * EMB)],
                         out_hbm.at[pl.ds(FULL_BLOCKS * VBLK * EMB, TAIL * EMB)],
                         so0).wait()


def _sc_kernel(x_hbm, table_hbm, pe_hbm, out_hbm,
               idx_v, in0, in1, in2, in3, ou0, ou1, ou2, ou3, pe_v,
               sg0, sg1, sg2, sg3, ss0, ss1, ss2, ss3):
    ins = (in0, in1, in2, in3)
    outs = (ou0, ou1, ou2, ou3)
    sgs = (sg0, sg1, sg2, sg3)
    sss = (ss0, ss1, ss2, ss3)

    wid = lax.axis_index("s") * NUM_CORES + lax.axis_index("c")
    pltpu.sync_copy(pe_hbm, pe_v)
    pltpu.sync_copy(x_hbm.at[pl.ds(wid * NCHUNK, NCHUNK)], idx_v)
    base = wid * PER_W

    def g_cp(c, b):
        return pltpu.make_async_copy(table_hbm.at[idx_v.at[c]], ins[b], sgs[b])

    def s_cp(c, b):
        dst = out_hbm.at[pl.ds((base + c * CHUNK) * EMB, CHUNK * EMB)]
        return pltpu.make_async_copy(outs[b], dst, sss[b])

    for b in range(NBUF):
        g_cp(b, b).start()

    def quad(g, _):
        for b in range(NBUF):
            c = NBUF * g + b
            g_cp(c, b).wait()

            @pl.when(g >= 1)
            def _():
                s_cp(c - NBUF, b).wait()

            p0 = lax.rem(c * CHUNK, SEQ) * EMB
            i_buf = ins[b]
            o_buf = outs[b]

            @plsc.parallel_loop(0, CHUNK, unroll=4)
            def row(r):
                for d in range(EMB // 16):
                    o_buf[pl.ds(r * EMB + d * 16, 16)] = (
                        i_buf[r, pl.ds(d * 16, 16)] * SCALE
                        + pe_v[pl.ds(p0 + r * EMB + d * 16, 16)])

            @pl.when(g < NCHUNK // NBUF - 1)
            def _():
                g_cp(c + NBUF, b).start()

            s_cp(c, b).start()
        return 0

    lax.fori_loop(0, NCHUNK // NBUF, quad, 0)
    for b in range(NBUF):
        s_cp(NCHUNK - NBUF + b, b).wait()


def _impl(x, table):
    mesh = plsc.VectorSubcoreMesh(core_axis_name="c", subcore_axis_name="s")

    t_lin = pl.kernel(
        _transpose_kernel,
        out_type=jax.ShapeDtypeStruct((VOCAB * EMB,), jnp.float32),
        mesh=mesh,
        compiler_params=pltpu.CompilerParams(use_tc_tiling_on_sc=True,
                                             needs_layout_passes=False),
        scratch_types=[
            pltpu.VMEM((EMB, SKEW), jnp.float32),
            pltpu.VMEM((EMB, SKEW), jnp.float32),
            pltpu.VMEM((VBLK * EMB,), jnp.float32),
            pltpu.VMEM((VBLK * EMB,), jnp.float32),
            pltpu.VMEM((TAIL, EMB), jnp.float32),
            pltpu.SemaphoreType.DMA,
            pltpu.SemaphoreType.DMA,
            pltpu.SemaphoreType.DMA,
            pltpu.SemaphoreType.DMA,
        ],
    )(table.T, table[VOCAB - TAIL:])

    xf = x.reshape(TOTAL // CHUNK, CHUNK)
    out = pl.kernel(
        _sc_kernel,
        out_type=jax.ShapeDtypeStruct((TOTAL * EMB,), jnp.float32),
        mesh=mesh,
        compiler_params=pltpu.CompilerParams(use_tc_tiling_on_sc=False),
        scratch_types=(
            [pltpu.VMEM((NCHUNK, CHUNK), jnp.int32)]
            + [pltpu.VMEM((CHUNK, EMB), jnp.float32) for _ in range(NBUF)]
            + [pltpu.VMEM((CHUNK * EMB,), jnp.float32) for _ in range(NBUF)]
            + [pltpu.VMEM((PE_ROWS * EMB,), jnp.float32)]
            + [pltpu.SemaphoreType.DMA] * (2 * NBUF)
        ),
    )(xf, t_lin.reshape(VOCAB, EMB), jnp.asarray(_PE2))
    return out.reshape(BATCH, SEQ, EMB)


def kernel(x, table):
    return _impl(x, table)


# speedup vs baseline: 1.4429x; 1.1572x over previous
import math

import jax
import jax.numpy as jnp
import numpy as np
from jax import lax
from jax.experimental import pallas as pl
from jax.experimental.pallas import tpu as pltpu
from jax.experimental.pallas import tpu_sc as plsc

VOCAB = 1000000
EMB = 64
SEQ = 200
BATCH = 4096

NUM_CORES = 2
NUM_SUBCORES = 16
NUM_WORKERS = NUM_CORES * NUM_SUBCORES

CHUNK = 128
TOTAL = BATCH * SEQ
PER_W = TOTAL // NUM_WORKERS
NCHUNK = PER_W // CHUNK
NBUF = 4
SCALE = math.sqrt(float(EMB))
PE_ROWS = 2 * SEQ


def _positional_encoding_np(max_len, d):
    position = np.arange(max_len)[:, None].astype(np.float32)
    div_term = np.exp(np.arange(0, d, 2).astype(np.float32) * (-np.log(10000.0) / d))
    pe = np.zeros((max_len, d), dtype=np.float32)
    pe[:, 0::2] = np.sin(position * div_term)
    pe[:, 1::2] = np.cos(position * div_term)
    return pe


_PE2 = np.concatenate([_positional_encoding_np(SEQ, EMB)] * 2, axis=0).reshape(-1)


def _sc_kernel(x_hbm, table_hbm, pe_hbm, out_hbm,
               idx_v, in0, in1, in2, in3, ou0, ou1, ou2, ou3, pe_v,
               sg0, sg1, sg2, sg3, ss0, ss1, ss2, ss3):
    ins = (in0, in1, in2, in3)
    outs = (ou0, ou1, ou2, ou3)
    sgs = (sg0, sg1, sg2, sg3)
    sss = (ss0, ss1, ss2, ss3)

    wid = lax.axis_index("s") * NUM_CORES + lax.axis_index("c")
    pltpu.sync_copy(pe_hbm, pe_v)
    pltpu.sync_copy(x_hbm.at[pl.ds(wid * NCHUNK, NCHUNK)], idx_v)
    base = wid * PER_W

    def g_cp(c, b):
        return pltpu.make_async_copy(table_hbm.at[idx_v.at[c]], ins[b], sgs[b])

    def s_cp(c, b):
        dst = out_hbm.at[pl.ds((base + c * CHUNK) * EMB, CHUNK * EMB)]
        return pltpu.make_async_copy(outs[b], dst, sss[b])

    for b in range(NBUF):
        g_cp(b, b).start()

    def quad(g, _):
        for b in range(NBUF):
            c = NBUF * g + b
            g_cp(c, b).wait()

            @pl.when(g >= 1)
            def _():
                s_cp(c - NBUF, b).wait()

            p0 = lax.rem(c * CHUNK, SEQ) * EMB
            i_buf = ins[b]
            o_buf = outs[b]

            @plsc.parallel_loop(0, CHUNK, unroll=4)
            def row(r):
                for d in range(EMB // 16):
                    o_buf[pl.ds(r * EMB + d * 16, 16)] = (
                        i_buf[r, pl.ds(d * 16, 16)] * SCALE
                        + pe_v[pl.ds(p0 + r * EMB + d * 16, 16)])

            @pl.when(g < NCHUNK // NBUF - 1)
            def _():
                g_cp(c + NBUF, b).start()

            s_cp(c, b).start()
        return 0

    lax.fori_loop(0, NCHUNK // NBUF, quad, 0)
    for b in range(NBUF):
        s_cp(NCHUNK - NBUF + b, b).wait()


def _impl(x, table):
    xf = x.reshape(TOTAL // CHUNK, CHUNK)
    mesh = plsc.VectorSubcoreMesh(core_axis_name="c", subcore_axis_name="s")
    out = pl.kernel(
        _sc_kernel,
        out_type=jax.ShapeDtypeStruct((TOTAL * EMB,), jnp.float32),
        mesh=mesh,
        compiler_params=pltpu.CompilerParams(use_tc_tiling_on_sc=False),
        scratch_types=(
            [pltpu.VMEM((NCHUNK, CHUNK), jnp.int32)]
            + [pltpu.VMEM((CHUNK, EMB), jnp.float32) for _ in range(NBUF)]
            + [pltpu.VMEM((CHUNK * EMB,), jnp.float32) for _ in range(NBUF)]
            + [pltpu.VMEM((PE_ROWS * EMB,), jnp.float32)]
            + [pltpu.SemaphoreType.DMA] * (2 * NBUF)
        ),
    )(xf, table, jnp.asarray(_PE2))
    return out.reshape(BATCH, SEQ, EMB)


def kernel(x, table):
    return _impl(x, table)
